# baseline (speedup 1.0000x reference)
import functools

import jax
import jax.numpy as jnp
from jax import lax
from jax.experimental import pallas as pl
from jax.experimental.pallas import tpu as pltpu
from jax.experimental.pallas import tpu_sc as plsc

N = 10000
D = 128
E = 320000
T = 8192
MARGIN_ = 0.1

N_PAD = 10240
E_PAD = 327680
NW = 32
EP = E_PAD // NW
CH = 128
NCH = EP // CH
NCH0 = 120
NCH1 = 2 * NCH - NCH0
NBUF = 2
NIDX = 4
ROWS_PER_TILE = N_PAD // 16
TCH = 128
TB = T // NW

_mesh = plsc.VectorSubcoreMesh(core_axis_name="c", subcore_axis_name="s")
_sc_params = pltpu.CompilerParams(needs_layout_passes=False)


@functools.partial(
    pl.kernel,
    out_type=(
        jax.ShapeDtypeStruct((2, N_PAD, D), jnp.float32),
        jax.ShapeDtypeStruct((NW, N_PAD), jnp.float32),
    ),
    mesh=_mesh,
    scratch_types=[
        pltpu.VMEM((NIDX, 2, CH), jnp.int32),
        [pltpu.VMEM((CH, D), jnp.float32)] * NBUF,
        pltpu.VMEM((N_PAD,), jnp.float32),
        pltpu.VMEM_SHARED((N_PAD, D), jnp.float32),
        [pltpu.SemaphoreType.DMA] * NIDX,
        [pltpu.SemaphoreType.DMA] * NBUF,
        [pltpu.SemaphoreType.DMA] * NBUF,
    ],
    compiler_params=_sc_params,
)
def _edge_agg(xpad, sd_i, out, out_deg, ring, rows, hist, agg_sh, si, sg, ss):
    c = lax.axis_index("c")
    s = lax.axis_index("s")
    wid = c * 16 + s
    zeros16 = jnp.zeros((16,), jnp.float32)
    ones16 = jnp.ones((16,), jnp.float32)

    def _zhist(i, _):
        hist[pl.ds(i * 16, 16)] = zeros16
        return 0

    lax.fori_loop(0, N_PAD // 16, _zhist, 0)

    def _zrow(i, _):
        for j in range(D // 16):
            rows[0][i, pl.ds(j * 16, 16)] = zeros16
        return 0

    lax.fori_loop(0, CH, _zrow, 0)
    for kk in range(ROWS_PER_TILE // CH):
        pltpu.sync_copy(rows[0], agg_sh.at[pl.ds(s * ROWS_PER_TILE + kk * CH, CH)])
    plsc.subcore_barrier()

    nch = jnp.where(c == 0, NCH0, NCH1)
    base = c * (16 * NCH0) + s * nch

    def _idx_start(slot, ck):
        pltpu.async_copy(sd_i.at[:, base + ck], ring.at[slot], si[slot])

    def _idx_wait(slot):
        pltpu.make_async_copy(sd_i.at[:, base], ring.at[slot], si[slot]).wait()

    def _gather_start(slot, rb):
        pltpu.async_copy(xpad.at[ring.at[slot, 0]], rows[rb], sg[rb])

    def _gather_wait(slot, rb):
        pltpu.make_async_copy(xpad.at[ring.at[slot, 0]], rows[rb], sg[rb]).wait()

    for b in range(NIDX):
        _idx_start(b, b)
    for b in range(NBUF):
        _idx_wait(b)
        _gather_start(b, b)

    def _step(t, _):
        for b in range(NIDX):
            ck = t * NIDX + b
            rb = b % NBUF
            _gather_wait(b, rb)
            sdesc = pltpu.async_copy(rows[rb], agg_sh.at[ring.at[b, 1]], ss[rb],
                                     add=True)
            for j in range(CH // 16):
                idx = ring[b, 1, pl.ds(j * 16, 16)]
                plsc.addupdate_scatter(hist, [idx], ones16)
            sdesc.wait()

            @pl.when(ck + NIDX < nch)
            def _():
                _idx_start(b, ck + NIDX)

            @pl.when(ck + NBUF < nch)
            def _():
                nslot = (b + NBUF) % NIDX
                _idx_wait(nslot)
                _gather_start(nslot, rb)
        return 0

    lax.fori_loop(0, nch // NIDX, _step, 0)
    plsc.subcore_barrier()

    pltpu.sync_copy(hist, out_deg.at[wid])
    for kk in range(ROWS_PER_TILE // CH):
        r0 = s * ROWS_PER_TILE + kk * CH
        pltpu.sync_copy(agg_sh.at[pl.ds(r0, CH)], out.at[c, pl.ds(r0, CH)])


def _sage_tc(x_ref, p_ref, dp_ref, ws_ref, wn_ref, b_ref, h_ref):
    a = p_ref[0] + p_ref[1]
    deg = jnp.maximum(jnp.sum(dp_ref[...], axis=0), 1.0)
    mean = a / deg[:, None]
    h = (x_ref[...] @ ws_ref[...] + mean @ wn_ref[...]) + b_ref[...]
    h = jnp.maximum(h, 0.0)
    norm = jnp.sqrt(jnp.sum(h * h, axis=1, keepdims=True)) + 1e-12
    h_ref[...] = h / norm


@functools.partial(
    pl.kernel,
    out_type=jax.ShapeDtypeStruct((T,), jnp.float32),
    mesh=_mesh,
    scratch_types=[
        pltpu.VMEM((TCH,), jnp.int32),
        pltpu.VMEM((TCH,), jnp.int32),
        pltpu.VMEM((TCH,), jnp.int32),
        pltpu.VMEM((TCH, D), jnp.float32),
        pltpu.VMEM((TCH, D), jnp.float32),
        pltpu.VMEM((TCH, D), jnp.float32),
        pltpu.VMEM((TCH,), jnp.float32),
        [pltpu.SemaphoreType.DMA] * 3,
    ],
    compiler_params=_sc_params,
)
def _triplet_score(h, src_i, dst_i, neg_i, out, si, di, ni, sr, dr, nr, ov, sem):
    c = lax.axis_index("c")
    s = lax.axis_index("s")
    wid = c * 16 + s

    def _chunk(t, _):
        base = wid * TB + t * TCH
        pltpu.sync_copy(src_i.at[pl.ds(base, TCH)], si)
        pltpu.sync_copy(dst_i.at[pl.ds(base, TCH)], di)
        pltpu.sync_copy(neg_i.at[pl.ds(base, TCH)], ni)
        d0 = pltpu.async_copy(h.at[si], sr, sem[0])
        d1 = pltpu.async_copy(h.at[di], dr, sem[1])
        d2 = pltpu.async_copy(h.at[ni], nr, sem[2])
        d0.wait()
        d1.wait()
        d2.wait()

        lane = lax.iota(jnp.int32, 16)

        def _grp(g, _):
            vec = jnp.zeros((16,), jnp.float32)
            for l in range(16):
                i = g * 16 + l
                ab = jnp.zeros((16,), jnp.float32)
                ac = jnp.zeros((16,), jnp.float32)
                for j in range(D // 16):
                    sv = sr[i, pl.ds(j * 16, 16)]
                    ab = ab + sv * dr[i, pl.ds(j * 16, 16)]
                    ac = ac + sv * nr[i, pl.ds(j * 16, 16)]
                sc = jnp.maximum(jnp.sum(ac) - jnp.sum(ab) + MARGIN_, 0.0)
                vec = jnp.where(lane == l, sc, vec)
            ov[pl.ds(g * 16, 16)] = vec
            return 0

        lax.fori_loop(0, TCH // 16, _grp, 0)
        pltpu.sync_copy(ov, out.at[pl.ds(base, TCH)])
        return 0

    lax.fori_loop(0, TB // TCH, _chunk, 0)


def kernel(x, edge_index, src, dst, neg, W_self, W_neigh, b):
    sd_i = jnp.pad(edge_index.astype(jnp.int32), ((0, 0), (0, E_PAD - E)),
                   constant_values=N).reshape(2, NW * NCH, CH)

    xpad = jnp.pad(x, ((0, N_PAD - N), (0, 0)))
    partials, deg_parts = _edge_agg(xpad, sd_i)

    BN = 512
    h = pl.pallas_call(
        _sage_tc,
        grid=(N_PAD // BN,),
        in_specs=[
            pl.BlockSpec((BN, D), lambda i: (i, 0)),
            pl.BlockSpec((2, BN, D), lambda i: (0, i, 0)),
            pl.BlockSpec((NW, BN), lambda i: (0, i)),
            pl.BlockSpec((D, D), lambda i: (0, 0)),
            pl.BlockSpec((D, D), lambda i: (0, 0)),
            pl.BlockSpec((1, D), lambda i: (0, 0)),
        ],
        out_specs=pl.BlockSpec((BN, D), lambda i: (i, 0)),
        out_shape=jax.ShapeDtypeStruct((N_PAD, D), jnp.float32),
    )(xpad, partials, deg_parts, W_self, W_neigh, b.reshape(1, D))

    return _triplet_score(
        h, src.astype(jnp.int32), dst.astype(jnp.int32), neg.astype(jnp.int32))


# speedup vs baseline: 1.1421x; 1.1421x over previous
import functools

import jax
import jax.numpy as jnp
from jax import lax
from jax.experimental import pallas as pl
from jax.experimental.pallas import tpu as pltpu
from jax.experimental.pallas import tpu_sc as plsc

N = 10000
D = 128
E = 320000
T = 8192
MARGIN_ = 0.1

N_PAD = 10240
E_PAD = 327680
NW = 32
EP = E_PAD // NW
CH = 128
NCH = EP // CH
NCH0 = 120
NCH1 = 2 * NCH - NCH0
NBUF = 2
NIDX = 4
ROWS_PER_TILE = N_PAD // 16
TCH = 128
TB = T // NW

_mesh = plsc.VectorSubcoreMesh(core_axis_name="c", subcore_axis_name="s")
_sc_params = pltpu.CompilerParams(needs_layout_passes=False)


@functools.partial(
    pl.kernel,
    out_type=(
        jax.ShapeDtypeStruct((2, N_PAD, D), jnp.float32),
        jax.ShapeDtypeStruct((NW, N_PAD), jnp.float32),
    ),
    mesh=_mesh,
    scratch_types=[
        pltpu.VMEM((NIDX, 2, CH), jnp.int32),
        [pltpu.VMEM((CH, D), jnp.float32)] * NBUF,
        pltpu.VMEM((N_PAD,), jnp.float32),
        pltpu.VMEM_SHARED((N_PAD, D), jnp.float32),
        [pltpu.SemaphoreType.DMA] * NIDX,
        [pltpu.SemaphoreType.DMA] * NBUF,
        [pltpu.SemaphoreType.DMA] * NBUF,
    ],
    compiler_params=_sc_params,
)
def _edge_agg(xpad, sd_i, out, out_deg, ring, rows, hist, agg_sh, si, sg, ss):
    c = lax.axis_index("c")
    s = lax.axis_index("s")
    wid = c * 16 + s
    zeros16 = jnp.zeros((16,), jnp.float32)
    ones16 = jnp.ones((16,), jnp.float32)

    def _zhist(i, _):
        hist[pl.ds(i * 16, 16)] = zeros16
        return 0

    lax.fori_loop(0, N_PAD // 16, _zhist, 0)

    def _zrow(i, _):
        for j in range(D // 16):
            rows[0][i, pl.ds(j * 16, 16)] = zeros16
        return 0

    lax.fori_loop(0, CH, _zrow, 0)
    for kk in range(ROWS_PER_TILE // CH):
        pltpu.sync_copy(rows[0], agg_sh.at[pl.ds(s * ROWS_PER_TILE + kk * CH, CH)])
    plsc.subcore_barrier()

    nch = jnp.where(c == 0, NCH0, NCH1)
    base = c * (16 * NCH0) + s * nch

    def _idx_start(slot, ck):
        pltpu.async_copy(sd_i.at[base + ck], ring.at[slot], si[slot])

    def _idx_wait(slot):
        pltpu.make_async_copy(sd_i.at[base], ring.at[slot], si[slot]).wait()

    def _gather_start(slot, rb):
        pltpu.async_copy(xpad.at[ring.at[slot, 0]], rows[rb], sg[rb])

    def _gather_wait(slot, rb):
        pltpu.make_async_copy(xpad.at[ring.at[slot, 0]], rows[rb], sg[rb]).wait()

    for b in range(NIDX):
        _idx_start(b, b)
    for b in range(NBUF):
        _idx_wait(b)
        _gather_start(b, b)

    def _step(t, _):
        for b in range(NIDX):
            ck = t * NIDX + b
            rb = b % NBUF
            _gather_wait(b, rb)
            sdesc = pltpu.async_copy(rows[rb], agg_sh.at[ring.at[b, 1]], ss[rb],
                                     add=True)
            for j in range(CH // 16):
                idx = ring[b, 1, pl.ds(j * 16, 16)]
                plsc.addupdate_scatter(hist, [idx], ones16)
            sdesc.wait()

            @pl.when(ck + NIDX < nch)
            def _():
                _idx_start(b, ck + NIDX)

            @pl.when(ck + NBUF < nch)
            def _():
                nslot = (b + NBUF) % NIDX
                _idx_wait(nslot)
                _gather_start(nslot, rb)
        return 0

    lax.fori_loop(0, nch // NIDX, _step, 0)
    plsc.subcore_barrier()

    pltpu.sync_copy(hist, out_deg.at[wid])
    for kk in range(ROWS_PER_TILE // CH):
        r0 = s * ROWS_PER_TILE + kk * CH
        pltpu.sync_copy(agg_sh.at[pl.ds(r0, CH)], out.at[c, pl.ds(r0, CH)])


def _sage_tc(x_ref, p_ref, dp_ref, ws_ref, wn_ref, b_ref, h_ref):
    a = p_ref[0] + p_ref[1]
    deg = jnp.maximum(jnp.sum(dp_ref[...], axis=0), 1.0)
    mean = a / deg[:, None]
    h = (x_ref[...] @ ws_ref[...] + mean @ wn_ref[...]) + b_ref[...]
    h = jnp.maximum(h, 0.0)
    norm = jnp.sqrt(jnp.sum(h * h, axis=1, keepdims=True)) + 1e-12
    h_ref[...] = h / norm


@functools.partial(
    pl.kernel,
    out_type=jax.ShapeDtypeStruct((T,), jnp.float32),
    mesh=_mesh,
    scratch_types=[
        pltpu.VMEM((TCH,), jnp.int32),
        pltpu.VMEM((TCH,), jnp.int32),
        pltpu.VMEM((TCH,), jnp.int32),
        pltpu.VMEM((TCH, D), jnp.float32),
        pltpu.VMEM((TCH, D), jnp.float32),
        pltpu.VMEM((TCH, D), jnp.float32),
        pltpu.VMEM((TCH,), jnp.float32),
        [pltpu.SemaphoreType.DMA] * 3,
    ],
    compiler_params=_sc_params,
)
def _triplet_score(h, src_i, dst_i, neg_i, out, si, di, ni, sr, dr, nr, ov, sem):
    c = lax.axis_index("c")
    s = lax.axis_index("s")
    wid = c * 16 + s

    def _chunk(t, _):
        base = wid * TB + t * TCH
        pltpu.sync_copy(src_i.at[pl.ds(base, TCH)], si)
        pltpu.sync_copy(dst_i.at[pl.ds(base, TCH)], di)
        pltpu.sync_copy(neg_i.at[pl.ds(base, TCH)], ni)
        d0 = pltpu.async_copy(h.at[si], sr, sem[0])
        d1 = pltpu.async_copy(h.at[di], dr, sem[1])
        d2 = pltpu.async_copy(h.at[ni], nr, sem[2])
        d0.wait()
        d1.wait()
        d2.wait()

        lane = lax.iota(jnp.int32, 16)

        def _grp(g, _):
            vec = jnp.zeros((16,), jnp.float32)
            for l in range(16):
                i = g * 16 + l
                ab = jnp.zeros((16,), jnp.float32)
                ac = jnp.zeros((16,), jnp.float32)
                for j in range(D // 16):
                    sv = sr[i, pl.ds(j * 16, 16)]
                    ab = ab + sv * dr[i, pl.ds(j * 16, 16)]
                    ac = ac + sv * nr[i, pl.ds(j * 16, 16)]
                sc = jnp.maximum(jnp.sum(ac) - jnp.sum(ab) + MARGIN_, 0.0)
                vec = jnp.where(lane == l, sc, vec)
            ov[pl.ds(g * 16, 16)] = vec
            return 0

        lax.fori_loop(0, TCH // 16, _grp, 0)
        pltpu.sync_copy(ov, out.at[pl.ds(base, TCH)])
        return 0

    lax.fori_loop(0, TB // TCH, _chunk, 0)


def kernel(x, edge_index, src, dst, neg, W_self, W_neigh, b):
    e_src = edge_index[0].astype(jnp.int32)
    e_dst = edge_index[1].astype(jnp.int32)
    pad = E_PAD - E
    e_src = jnp.concatenate([e_src, jnp.zeros((pad,), jnp.int32)])
    e_dst = jnp.concatenate([e_dst, jnp.full((pad,), N, jnp.int32)])
    sd_i = jnp.concatenate(
        [e_src.reshape(NW * NCH, 1, CH), e_dst.reshape(NW * NCH, 1, CH)],
        axis=1)

    xpad = jnp.pad(x, ((0, N_PAD - N), (0, 0)))
    partials, deg_parts = _edge_agg(xpad, sd_i)

    BN = 512
    h = pl.pallas_call(
        _sage_tc,
        grid=(N_PAD // BN,),
        in_specs=[
            pl.BlockSpec((BN, D), lambda i: (i, 0)),
            pl.BlockSpec((2, BN, D), lambda i: (0, i, 0)),
            pl.BlockSpec((NW, BN), lambda i: (0, i)),
            pl.BlockSpec((D, D), lambda i: (0, 0)),
            pl.BlockSpec((D, D), lambda i: (0, 0)),
            pl.BlockSpec((1, D), lambda i: (0, 0)),
        ],
        out_specs=pl.BlockSpec((BN, D), lambda i: (i, 0)),
        out_shape=jax.ShapeDtypeStruct((N_PAD, D), jnp.float32),
    )(xpad, partials, deg_parts, W_self, W_neigh, b.reshape(1, D))

    return _triplet_score(
        h, src.astype(jnp.int32), dst.astype(jnp.int32), neg.astype(jnp.int32))
